# Initial kernel scaffold; baseline (speedup 1.0000x reference)
import jax
import jax.numpy as jnp
from jax.experimental import pallas as pl


def kernel(x, W1, b1, W2, b2, W3, b3, W4, b4, W5, b5, W6, b6, emb):
    raise NotImplementedError("write your pallas kernel here")


# speedup vs baseline: 1.8576x; 1.8576x over previous
import functools

import jax
import jax.numpy as jnp
from jax.experimental import pallas as pl

B = 32768
FEATURE_DIM = 256
LATENT_DIM = 64
K = 1024
COMMITMENT_COST = 0.25

BLK = 1024


def _fused_kernel(x_ref, w1_ref, b1_ref, w2_ref, b2_ref, w3_ref, b3_ref,
                  w4_ref, b4_ref, w5_ref, b5_ref, w6_ref, b6_ref, emb_ref,
                  z_out_ref, qsum_ref, rsum_ref):
    x = x_ref[...]
    h = jax.nn.relu(jnp.dot(x, w1_ref[...], preferred_element_type=jnp.float32)
                    + b1_ref[...])
    h = jax.nn.relu(jnp.dot(h, w2_ref[...], preferred_element_type=jnp.float32)
                    + b2_ref[...])
    z_e = jax.nn.relu(jnp.dot(h, w3_ref[...], preferred_element_type=jnp.float32)
                      + b3_ref[...])
    emb = emb_ref[...]
    cross = jax.lax.dot_general(z_e, emb, (((1,), (1,)), ((), ())),
                                preferred_element_type=jnp.float32)
    d = (jnp.sum(z_e * z_e, axis=1, keepdims=True)
         - 2.0 * cross
         + jnp.sum(emb * emb, axis=1)[None, :])
    dmin = jnp.min(d, axis=1, keepdims=True)
    iota = jax.lax.broadcasted_iota(jnp.int32, d.shape, 1)
    idx = jnp.min(jnp.where(d == dmin, iota, K), axis=1)
    onehot = (iota == idx[:, None]).astype(jnp.float32)
    z_q = jnp.dot(onehot, emb, preferred_element_type=jnp.float32)
    qdiff = z_q - z_e
    z_q_st = z_e + qdiff
    z_out_ref[...] = z_q_st
    g = jax.nn.relu(jnp.dot(z_q_st, w4_ref[...], preferred_element_type=jnp.float32)
                    + b4_ref[...])
    g = jax.nn.relu(jnp.dot(g, w5_ref[...], preferred_element_type=jnp.float32)
                    + b5_ref[...])
    x_recon = jax.nn.relu(jnp.dot(g, w6_ref[...], preferred_element_type=jnp.float32)
                          + b6_ref[...])
    rdiff = x_recon - x
    qs = jnp.sum(qdiff * qdiff, axis=0, keepdims=True)
    rs = jnp.sum(rdiff * rdiff, axis=0, keepdims=True)

    @pl.when(pl.program_id(0) == 0)
    def _init():
        qsum_ref[...] = qs
        rsum_ref[...] = rs

    @pl.when(pl.program_id(0) != 0)
    def _acc():
        qsum_ref[...] += qs
        rsum_ref[...] += rs


@functools.partial(jax.jit, static_argnames=())
def kernel(x, W1, b1, W2, b2, W3, b3, W4, b4, W5, b5, W6, b6, emb):
    grid = B // BLK
    full = lambda shape: pl.BlockSpec(shape, lambda i: (0,) * len(shape))
    z_latent, qsum, rsum = pl.pallas_call(
        _fused_kernel,
        grid=(grid,),
        in_specs=[
            pl.BlockSpec((BLK, FEATURE_DIM), lambda i: (i, 0)),
            full((FEATURE_DIM, 64)), full((64,)),
            full((64, 128)), full((128,)),
            full((128, LATENT_DIM)), full((LATENT_DIM,)),
            full((LATENT_DIM, 128)), full((128,)),
            full((128, 64)), full((64,)),
            full((64, FEATURE_DIM)), full((FEATURE_DIM,)),
            full((K, LATENT_DIM)),
        ],
        out_specs=[
            pl.BlockSpec((BLK, LATENT_DIM), lambda i: (i, 0)),
            pl.BlockSpec((1, LATENT_DIM), lambda i: (0, 0)),
            pl.BlockSpec((1, FEATURE_DIM), lambda i: (0, 0)),
        ],
        out_shape=[
            jax.ShapeDtypeStruct((B, LATENT_DIM), jnp.float32),
            jax.ShapeDtypeStruct((1, LATENT_DIM), jnp.float32),
            jax.ShapeDtypeStruct((1, FEATURE_DIM), jnp.float32),
        ],
    )(x, W1, b1, W2, b2, W3, b3, W4, b4, W5, b5, W6, b6, emb)
    quant_loss = (1.0 + COMMITMENT_COST) * (jnp.sum(qsum) / (B * LATENT_DIM))
    recon_loss = jnp.sum(rsum) / (B * FEATURE_DIM)
    loss = recon_loss + quant_loss
    return (z_latent, loss)


# speedup vs baseline: 2.0329x; 1.0944x over previous
import functools

import jax
import jax.numpy as jnp
from jax.experimental import pallas as pl

B = 32768
FEATURE_DIM = 256
LATENT_DIM = 64
K = 1024
COMMITMENT_COST = 0.25

BLK = 2048


def _fused_kernel(x_ref, w1_ref, b1_ref, w2_ref, b2_ref, w3_ref, b3_ref,
                  w4_ref, b4_ref, w5_ref, b5_ref, w6_ref, b6_ref, emb_ref,
                  z_out_ref, qsum_ref, rsum_ref):
    x = x_ref[...]
    h = jax.nn.relu(jnp.dot(x, w1_ref[...], preferred_element_type=jnp.float32)
                    + b1_ref[...])
    h = jax.nn.relu(jnp.dot(h, w2_ref[...], preferred_element_type=jnp.float32)
                    + b2_ref[...])
    z_e = jax.nn.relu(jnp.dot(h, w3_ref[...], preferred_element_type=jnp.float32)
                      + b3_ref[...])
    emb = emb_ref[...]
    cross = jax.lax.dot_general(z_e, emb, (((1,), (1,)), ((), ())),
                                preferred_element_type=jnp.float32)
    d = (jnp.sum(z_e * z_e, axis=1, keepdims=True)
         - 2.0 * cross
         + jnp.sum(emb * emb, axis=1)[None, :])
    dmin = jnp.min(d, axis=1, keepdims=True)
    iota = jax.lax.broadcasted_iota(jnp.int32, d.shape, 1)
    idx = jnp.min(jnp.where(d == dmin, iota, K), axis=1)
    onehot = (iota == idx[:, None]).astype(jnp.float32)
    z_q = jnp.dot(onehot, emb, preferred_element_type=jnp.float32)
    qdiff = z_q - z_e
    z_q_st = z_e + qdiff
    z_out_ref[...] = z_q_st
    g = jax.nn.relu(jnp.dot(z_q_st, w4_ref[...], preferred_element_type=jnp.float32)
                    + b4_ref[...])
    g = jax.nn.relu(jnp.dot(g, w5_ref[...], preferred_element_type=jnp.float32)
                    + b5_ref[...])
    x_recon = jax.nn.relu(jnp.dot(g, w6_ref[...], preferred_element_type=jnp.float32)
                          + b6_ref[...])
    rdiff = x_recon - x
    qs = jnp.sum(qdiff * qdiff, axis=0, keepdims=True)
    rs = jnp.sum(rdiff * rdiff, axis=0, keepdims=True)

    @pl.when(pl.program_id(0) == 0)
    def _init():
        qsum_ref[...] = qs
        rsum_ref[...] = rs

    @pl.when(pl.program_id(0) != 0)
    def _acc():
        qsum_ref[...] += qs
        rsum_ref[...] += rs


@functools.partial(jax.jit, static_argnames=())
def kernel(x, W1, b1, W2, b2, W3, b3, W4, b4, W5, b5, W6, b6, emb):
    grid = B // BLK
    full = lambda shape: pl.BlockSpec(shape, lambda i: (0,) * len(shape))
    z_latent, qsum, rsum = pl.pallas_call(
        _fused_kernel,
        grid=(grid,),
        in_specs=[
            pl.BlockSpec((BLK, FEATURE_DIM), lambda i: (i, 0)),
            full((FEATURE_DIM, 64)), full((64,)),
            full((64, 128)), full((128,)),
            full((128, LATENT_DIM)), full((LATENT_DIM,)),
            full((LATENT_DIM, 128)), full((128,)),
            full((128, 64)), full((64,)),
            full((64, FEATURE_DIM)), full((FEATURE_DIM,)),
            full((K, LATENT_DIM)),
        ],
        out_specs=[
            pl.BlockSpec((BLK, LATENT_DIM), lambda i: (i, 0)),
            pl.BlockSpec((1, LATENT_DIM), lambda i: (0, 0)),
            pl.BlockSpec((1, FEATURE_DIM), lambda i: (0, 0)),
        ],
        out_shape=[
            jax.ShapeDtypeStruct((B, LATENT_DIM), jnp.float32),
            jax.ShapeDtypeStruct((1, LATENT_DIM), jnp.float32),
            jax.ShapeDtypeStruct((1, FEATURE_DIM), jnp.float32),
        ],
    )(x, W1, b1, W2, b2, W3, b3, W4, b4, W5, b5, W6, b6, emb)
    quant_loss = (1.0 + COMMITMENT_COST) * (jnp.sum(qsum) / (B * LATENT_DIM))
    recon_loss = jnp.sum(rsum) / (B * FEATURE_DIM)
    loss = recon_loss + quant_loss
    return (z_latent, loss)


# speedup vs baseline: 2.1036x; 1.0347x over previous
import functools

import jax
import jax.numpy as jnp
from jax.experimental import pallas as pl

B = 32768
FEATURE_DIM = 256
LATENT_DIM = 64
K = 1024
COMMITMENT_COST = 0.25

BLK = 4096


def _fused_kernel(x_ref, w1_ref, b1_ref, w2_ref, b2_ref, w3_ref, b3_ref,
                  w4_ref, b4_ref, w5_ref, b5_ref, w6_ref, b6_ref, emb_ref,
                  z_out_ref, qsum_ref, rsum_ref):
    x = x_ref[...]
    h = jax.nn.relu(jnp.dot(x, w1_ref[...], preferred_element_type=jnp.float32)
                    + b1_ref[...])
    h = jax.nn.relu(jnp.dot(h, w2_ref[...], preferred_element_type=jnp.float32)
                    + b2_ref[...])
    z_e = jax.nn.relu(jnp.dot(h, w3_ref[...], preferred_element_type=jnp.float32)
                      + b3_ref[...])
    emb = emb_ref[...]
    cross = jax.lax.dot_general(z_e, emb, (((1,), (1,)), ((), ())),
                                preferred_element_type=jnp.float32)
    d = (jnp.sum(z_e * z_e, axis=1, keepdims=True)
         - 2.0 * cross
         + jnp.sum(emb * emb, axis=1)[None, :])
    dmin = jnp.min(d, axis=1, keepdims=True)
    iota = jax.lax.broadcasted_iota(jnp.int32, d.shape, 1)
    idx = jnp.min(jnp.where(d == dmin, iota, K), axis=1)
    onehot = (iota == idx[:, None]).astype(jnp.float32)
    z_q = jnp.dot(onehot, emb, preferred_element_type=jnp.float32)
    qdiff = z_q - z_e
    z_q_st = z_e + qdiff
    z_out_ref[...] = z_q_st
    g = jax.nn.relu(jnp.dot(z_q_st, w4_ref[...], preferred_element_type=jnp.float32)
                    + b4_ref[...])
    g = jax.nn.relu(jnp.dot(g, w5_ref[...], preferred_element_type=jnp.float32)
                    + b5_ref[...])
    x_recon = jax.nn.relu(jnp.dot(g, w6_ref[...], preferred_element_type=jnp.float32)
                          + b6_ref[...])
    rdiff = x_recon - x
    qs = jnp.sum(qdiff * qdiff, axis=0, keepdims=True)
    rs = jnp.sum(rdiff * rdiff, axis=0, keepdims=True)

    @pl.when(pl.program_id(0) == 0)
    def _init():
        qsum_ref[...] = qs
        rsum_ref[...] = rs

    @pl.when(pl.program_id(0) != 0)
    def _acc():
        qsum_ref[...] += qs
        rsum_ref[...] += rs


@functools.partial(jax.jit, static_argnames=())
def kernel(x, W1, b1, W2, b2, W3, b3, W4, b4, W5, b5, W6, b6, emb):
    grid = B // BLK
    full = lambda shape: pl.BlockSpec(shape, lambda i: (0,) * len(shape))
    z_latent, qsum, rsum = pl.pallas_call(
        _fused_kernel,
        grid=(grid,),
        in_specs=[
            pl.BlockSpec((BLK, FEATURE_DIM), lambda i: (i, 0)),
            full((FEATURE_DIM, 64)), full((64,)),
            full((64, 128)), full((128,)),
            full((128, LATENT_DIM)), full((LATENT_DIM,)),
            full((LATENT_DIM, 128)), full((128,)),
            full((128, 64)), full((64,)),
            full((64, FEATURE_DIM)), full((FEATURE_DIM,)),
            full((K, LATENT_DIM)),
        ],
        out_specs=[
            pl.BlockSpec((BLK, LATENT_DIM), lambda i: (i, 0)),
            pl.BlockSpec((1, LATENT_DIM), lambda i: (0, 0)),
            pl.BlockSpec((1, FEATURE_DIM), lambda i: (0, 0)),
        ],
        out_shape=[
            jax.ShapeDtypeStruct((B, LATENT_DIM), jnp.float32),
            jax.ShapeDtypeStruct((1, LATENT_DIM), jnp.float32),
            jax.ShapeDtypeStruct((1, FEATURE_DIM), jnp.float32),
        ],
    )(x, W1, b1, W2, b2, W3, b3, W4, b4, W5, b5, W6, b6, emb)
    quant_loss = (1.0 + COMMITMENT_COST) * (jnp.sum(qsum) / (B * LATENT_DIM))
    recon_loss = jnp.sum(rsum) / (B * FEATURE_DIM)
    loss = recon_loss + quant_loss
    return (z_latent, loss)


# speedup vs baseline: 2.1181x; 1.0069x over previous
import functools

import jax
import jax.numpy as jnp
from jax.experimental import pallas as pl
from jax.experimental.pallas import tpu as pltpu

B = 32768
FEATURE_DIM = 256
LATENT_DIM = 64
K = 1024
COMMITMENT_COST = 0.25

BLK = 4096


def _fused_kernel(x_ref, w1_ref, b1_ref, w2_ref, b2_ref, w3_ref, b3_ref,
                  w4_ref, b4_ref, w5_ref, b5_ref, w6_ref, b6_ref, emb_ref,
                  z_out_ref, qsum_ref, rsum_ref):
    x = x_ref[...]
    h = jax.nn.relu(jnp.dot(x, w1_ref[...], preferred_element_type=jnp.float32)
                    + b1_ref[...])
    h = jax.nn.relu(jnp.dot(h, w2_ref[...], preferred_element_type=jnp.float32)
                    + b2_ref[...])
    z_e = jax.nn.relu(jnp.dot(h, w3_ref[...], preferred_element_type=jnp.float32)
                      + b3_ref[...])
    emb = emb_ref[...]
    cross = jax.lax.dot_general(z_e, emb, (((1,), (1,)), ((), ())),
                                preferred_element_type=jnp.float32)
    d = (jnp.sum(z_e * z_e, axis=1, keepdims=True)
         - 2.0 * cross
         + jnp.sum(emb * emb, axis=1)[None, :])
    dmin = jnp.min(d, axis=1, keepdims=True)
    iota = jax.lax.broadcasted_iota(jnp.int32, d.shape, 1)
    idx = jnp.min(jnp.where(d == dmin, iota, K), axis=1)
    onehot = (iota == idx[:, None]).astype(jnp.float32)
    z_q = jnp.dot(onehot, emb, preferred_element_type=jnp.float32)
    qdiff = z_q - z_e
    z_q_st = z_e + qdiff
    z_out_ref[...] = z_q_st
    g = jax.nn.relu(jnp.dot(z_q_st, w4_ref[...], preferred_element_type=jnp.float32)
                    + b4_ref[...])
    g = jax.nn.relu(jnp.dot(g, w5_ref[...], preferred_element_type=jnp.float32)
                    + b5_ref[...])
    x_recon = jax.nn.relu(jnp.dot(g, w6_ref[...], preferred_element_type=jnp.float32)
                          + b6_ref[...])
    rdiff = x_recon - x
    qsum_ref[...] = jnp.sum(qdiff * qdiff, axis=0, keepdims=True)[None]
    rsum_ref[...] = jnp.sum(rdiff * rdiff, axis=0, keepdims=True)[None]


@functools.partial(jax.jit, static_argnames=())
def kernel(x, W1, b1, W2, b2, W3, b3, W4, b4, W5, b5, W6, b6, emb):
    grid = B // BLK
    full = lambda shape: pl.BlockSpec(shape, lambda i: (0,) * len(shape))
    z_latent, qsum, rsum = pl.pallas_call(
        _fused_kernel,
        grid=(grid,),
        in_specs=[
            pl.BlockSpec((BLK, FEATURE_DIM), lambda i: (i, 0)),
            full((FEATURE_DIM, 64)), full((64,)),
            full((64, 128)), full((128,)),
            full((128, LATENT_DIM)), full((LATENT_DIM,)),
            full((LATENT_DIM, 128)), full((128,)),
            full((128, 64)), full((64,)),
            full((64, FEATURE_DIM)), full((FEATURE_DIM,)),
            full((K, LATENT_DIM)),
        ],
        out_specs=[
            pl.BlockSpec((BLK, LATENT_DIM), lambda i: (i, 0)),
            pl.BlockSpec((1, 1, LATENT_DIM), lambda i: (i, 0, 0)),
            pl.BlockSpec((1, 1, FEATURE_DIM), lambda i: (i, 0, 0)),
        ],
        out_shape=[
            jax.ShapeDtypeStruct((B, LATENT_DIM), jnp.float32),
            jax.ShapeDtypeStruct((grid, 1, LATENT_DIM), jnp.float32),
            jax.ShapeDtypeStruct((grid, 1, FEATURE_DIM), jnp.float32),
        ],
        compiler_params=pltpu.CompilerParams(
            dimension_semantics=("parallel",)),
    )(x, W1, b1, W2, b2, W3, b3, W4, b4, W5, b5, W6, b6, emb)
    quant_loss = (1.0 + COMMITMENT_COST) * (jnp.sum(qsum) / (B * LATENT_DIM))
    recon_loss = jnp.sum(rsum) / (B * FEATURE_DIM)
    loss = recon_loss + quant_loss
    return (z_latent, loss)


# speedup vs baseline: 2.1580x; 1.0189x over previous
import functools

import jax
import jax.numpy as jnp
from jax.experimental import pallas as pl
from jax.experimental.pallas import tpu as pltpu

B = 32768
FEATURE_DIM = 256
LATENT_DIM = 64
K = 1024
COMMITMENT_COST = 0.25

BLK = 4096


def _fused_kernel(x_ref, w1_ref, b1_ref, w2_ref, b2_ref, w3_ref, b3_ref,
                  w4_ref, b4_ref, w5_ref, b5_ref, w6_ref, b6_ref, emb_ref,
                  z_out_ref, qsum_ref, rsum_ref):
    x = x_ref[...]
    h = jax.nn.relu(jnp.dot(x, w1_ref[...], preferred_element_type=jnp.float32)
                    + b1_ref[...])
    h = jax.nn.relu(jnp.dot(h, w2_ref[...], preferred_element_type=jnp.float32)
                    + b2_ref[...])
    z_e = jax.nn.relu(jnp.dot(h, w3_ref[...], preferred_element_type=jnp.float32)
                      + b3_ref[...])
    emb = emb_ref[...]
    cross = jax.lax.dot_general(z_e, emb, (((1,), (1,)), ((), ())),
                                preferred_element_type=jnp.float32)
    d = (jnp.sum(z_e * z_e, axis=1, keepdims=True)
         - 2.0 * cross
         + jnp.sum(emb * emb, axis=1)[None, :])
    idx = jnp.argmin(d, axis=1)
    iota = jax.lax.broadcasted_iota(jnp.int32, d.shape, 1)
    onehot = jnp.where(iota == idx[:, None], 1.0, 0.0)
    z_q = jnp.dot(onehot, emb, preferred_element_type=jnp.float32)
    qdiff = z_q - z_e
    z_q_st = z_e + qdiff
    z_out_ref[...] = z_q_st
    g = jax.nn.relu(jnp.dot(z_q_st, w4_ref[...], preferred_element_type=jnp.float32)
                    + b4_ref[...])
    g = jax.nn.relu(jnp.dot(g, w5_ref[...], preferred_element_type=jnp.float32)
                    + b5_ref[...])
    x_recon = jax.nn.relu(jnp.dot(g, w6_ref[...], preferred_element_type=jnp.float32)
                          + b6_ref[...])
    rdiff = x_recon - x
    qsum_ref[...] = jnp.sum(qdiff * qdiff, axis=0, keepdims=True)[None]
    rsum_ref[...] = jnp.sum(rdiff * rdiff, axis=0, keepdims=True)[None]


@functools.partial(jax.jit, static_argnames=())
def kernel(x, W1, b1, W2, b2, W3, b3, W4, b4, W5, b5, W6, b6, emb):
    grid = B // BLK
    full = lambda shape: pl.BlockSpec(shape, lambda i: (0,) * len(shape))
    z_latent, qsum, rsum = pl.pallas_call(
        _fused_kernel,
        grid=(grid,),
        in_specs=[
            pl.BlockSpec((BLK, FEATURE_DIM), lambda i: (i, 0)),
            full((FEATURE_DIM, 64)), full((64,)),
            full((64, 128)), full((128,)),
            full((128, LATENT_DIM)), full((LATENT_DIM,)),
            full((LATENT_DIM, 128)), full((128,)),
            full((128, 64)), full((64,)),
            full((64, FEATURE_DIM)), full((FEATURE_DIM,)),
            full((K, LATENT_DIM)),
        ],
        out_specs=[
            pl.BlockSpec((BLK, LATENT_DIM), lambda i: (i, 0)),
            pl.BlockSpec((1, 1, LATENT_DIM), lambda i: (i, 0, 0)),
            pl.BlockSpec((1, 1, FEATURE_DIM), lambda i: (i, 0, 0)),
        ],
        out_shape=[
            jax.ShapeDtypeStruct((B, LATENT_DIM), jnp.float32),
            jax.ShapeDtypeStruct((grid, 1, LATENT_DIM), jnp.float32),
            jax.ShapeDtypeStruct((grid, 1, FEATURE_DIM), jnp.float32),
        ],
        compiler_params=pltpu.CompilerParams(
            dimension_semantics=("parallel",)),
    )(x, W1, b1, W2, b2, W3, b3, W4, b4, W5, b5, W6, b6, emb)
    quant_loss = (1.0 + COMMITMENT_COST) * (jnp.sum(qsum) / (B * LATENT_DIM))
    recon_loss = jnp.sum(rsum) / (B * FEATURE_DIM)
    loss = recon_loss + quant_loss
    return (z_latent, loss)
